# baseline (speedup 1.0000x reference)
import jax
import jax.numpy as jnp
from jax import lax
from jax.experimental import pallas as pl
from jax.experimental.pallas import tpu as pltpu
from jax.experimental.pallas import tpu_sc as plsc

H = 56
W = 56
HW = H * W
NBLK = HW // 16
R = 64 * 256
NW = 32
ROWS_PER_W = R // NW
CHUNK = 8
NCHUNK = ROWS_PER_W // CHUNK
HALF = 3


def _body(x_hbm, t_hbm, out_hbm, tbuf, buf, redf, redi):
    cid = lax.axis_index("c")
    sid = lax.axis_index("s")
    wid = sid * 2 + cid
    base = wid * ROWS_PER_W

    lane = lax.iota(jnp.int32, 16)
    zerov = jnp.zeros((16,), jnp.float32)

    pltpu.sync_copy(t_hbm.at[pl.ds(base, ROWS_PER_W)],
                    tbuf.at[pl.ds(0, ROWS_PER_W)])

    redf[pl.ds(16, 16)] = jnp.full((16,), -jnp.inf, jnp.float32)
    redi[pl.ds(16, 16)] = jnp.full((16,), HW, jnp.int32)

    def process_row(roff):
        def amax_body(j, carry):
            m, bj = carry
            v = buf[pl.ds(roff + j * 16, 16)]
            gt = v > m
            return jnp.where(gt, v, m), jnp.where(gt, j, bj)

        m0 = jnp.full((16,), -jnp.inf, jnp.float32)
        b0 = jnp.zeros((16,), jnp.int32)
        m, bj = lax.fori_loop(0, NBLK, amax_body, (m0, b0))

        redf[pl.ds(0, 16)] = m
        redi[pl.ds(0, 16)] = bj * 16 + lane
        for sh in (8, 4, 2, 1):
            am = redf[pl.ds(0, 16)]
            ai = redi[pl.ds(0, 16)]
            bm = redf[pl.ds(sh, 16)]
            bi = redi[pl.ds(sh, 16)]
            better = (bm > am) | ((bm == am) & (bi < ai))
            redf[pl.ds(0, 16)] = jnp.where(better, bm, am)
            redi[pl.ds(0, 16)] = jnp.where(better, bi, ai)
        idx = redi[pl.ds(0, 16)][0]

        mh = idx // W
        mw = idx - mh * W
        h1 = jnp.maximum(mh - HALF, 0)
        h2 = jnp.minimum(mh + HALF, H - 1)
        w1 = jnp.maximum(mw - HALF, 0)
        w2 = jnp.minimum(mw + HALF, W - 1)
        area = (h2 - h1 + 1) * (w2 - w1 + 1)
        area_v = jnp.full((16,), 1.0, jnp.float32) * area.astype(jnp.float32)
        lamv = jnp.float32(HW) / (jnp.float32(HW) - area_v)

        def scale_body(j, _):
            sl = pl.ds(roff + j * 16, 16)
            buf[sl] = buf[sl] * lamv
            return 0

        lax.fori_loop(0, NBLK, scale_body, 0)

        msk = lane <= (w2 - w1)

        def zero_body(hr, _):
            sl = pl.ds(roff + hr * W + w1, 16)
            buf[sl] = jnp.where(msk, zerov, buf[sl])
            return 0

        lax.fori_loop(h1, h2 + 1, zero_body, 0)

    def chunk_body(ci, _):
        ebase = (base + ci * CHUNK) * HW
        pltpu.sync_copy(x_hbm.at[pl.ds(ebase, CHUNK * HW)],
                        buf.at[pl.ds(0, CHUNK * HW)])

        tv = tbuf[pl.ds(ci * CHUNK, 16)]
        for r in range(CHUNK):
            t = tv[r]

            @pl.when(t != 0.0)
            def _(roff=r * HW):
                process_row(roff)

        pltpu.sync_copy(buf.at[pl.ds(0, CHUNK * HW)],
                        out_hbm.at[pl.ds(ebase, CHUNK * HW)])
        return 0

    lax.fori_loop(0, NCHUNK, chunk_body, 0)


def kernel(x, T):
    b, c, h, w = x.shape
    x1 = x.reshape(R * HW)
    t1 = T.reshape(R)
    mesh = plsc.VectorSubcoreMesh(core_axis_name="c", subcore_axis_name="s")
    out = pl.kernel(
        _body,
        out_type=jax.ShapeDtypeStruct((R * HW,), jnp.float32),
        mesh=mesh,
        scratch_types=[
            pltpu.VMEM((ROWS_PER_W + 16,), jnp.float32),
            pltpu.VMEM((CHUNK * HW + 16,), jnp.float32),
            pltpu.VMEM((32,), jnp.float32),
            pltpu.VMEM((32,), jnp.int32),
        ],
    )(x1, t1)
    return out.reshape(b, c, h, w)


# speedup vs baseline: 1.2091x; 1.2091x over previous
import jax
import jax.numpy as jnp
from jax import lax
from jax.experimental import pallas as pl
from jax.experimental.pallas import tpu as pltpu
from jax.experimental.pallas import tpu_sc as plsc

H = 56
W = 56
HW = H * W
NBLK = HW // 16
R = 64 * 256
NW = 32
ROWS_PER_W = R // NW
CHUNK = 8
NCHUNK = ROWS_PER_W // CHUNK
HALF = 3


def _body(x_hbm, t_hbm, out_hbm, tbuf, buf, redf, redi):
    cid = lax.axis_index("c")
    sid = lax.axis_index("s")
    wid = sid * 2 + cid
    base = wid * ROWS_PER_W

    lane = lax.iota(jnp.int32, 16)
    zerov = jnp.zeros((16,), jnp.float32)

    pltpu.sync_copy(t_hbm.at[pl.ds(base, ROWS_PER_W)],
                    tbuf.at[pl.ds(0, ROWS_PER_W)])

    redf[pl.ds(16, 16)] = jnp.full((16,), -jnp.inf, jnp.float32)
    redi[pl.ds(16, 16)] = jnp.full((16,), HW, jnp.int32)

    def process_row(roff):
        def amax_body(j, carry):
            m, bj = carry
            v = buf[pl.ds(roff + j * 16, 16)]
            gt = v > m
            return jnp.where(gt, v, m), jnp.where(gt, j, bj)

        m0 = jnp.full((16,), -jnp.inf, jnp.float32)
        b0 = jnp.zeros((16,), jnp.int32)
        m, bj = lax.fori_loop(0, NBLK, amax_body, (m0, b0), unroll=14)

        redf[pl.ds(0, 16)] = m
        redi[pl.ds(0, 16)] = bj * 16 + lane
        for sh in (8, 4, 2, 1):
            am = redf[pl.ds(0, 16)]
            ai = redi[pl.ds(0, 16)]
            bm = redf[pl.ds(sh, 16)]
            bi = redi[pl.ds(sh, 16)]
            better = (bm > am) | ((bm == am) & (bi < ai))
            redf[pl.ds(0, 16)] = jnp.where(better, bm, am)
            redi[pl.ds(0, 16)] = jnp.where(better, bi, ai)
        idx = redi[pl.ds(0, 16)][0]

        mh = idx // W
        mw = idx - mh * W
        h1 = jnp.maximum(mh - HALF, 0)
        h2 = jnp.minimum(mh + HALF, H - 1)
        w1 = jnp.maximum(mw - HALF, 0)
        w2 = jnp.minimum(mw + HALF, W - 1)
        area = (h2 - h1 + 1) * (w2 - w1 + 1)
        area_v = jnp.full((16,), 1.0, jnp.float32) * area.astype(jnp.float32)
        lamv = jnp.float32(HW) / (jnp.float32(HW) - area_v)

        def scale_body(j, _):
            sl = pl.ds(roff + j * 16, 16)
            buf[sl] = buf[sl] * lamv
            return 0

        lax.fori_loop(0, NBLK, scale_body, 0, unroll=14)

        msk = lane <= (w2 - w1)

        def zero_body(hr, _):
            sl = pl.ds(roff + hr * W + w1, 16)
            buf[sl] = jnp.where(msk, zerov, buf[sl])
            return 0

        lax.fori_loop(h1, h2 + 1, zero_body, 0)

    def chunk_body(ci, _):
        ebase = (base + ci * CHUNK) * HW
        pltpu.sync_copy(x_hbm.at[pl.ds(ebase, CHUNK * HW)],
                        buf.at[pl.ds(0, CHUNK * HW)])

        tv = tbuf[pl.ds(ci * CHUNK, 16)]
        for r in range(CHUNK):
            t = tv[r]

            @pl.when(t != 0.0)
            def _(roff=r * HW):
                process_row(roff)

        pltpu.sync_copy(buf.at[pl.ds(0, CHUNK * HW)],
                        out_hbm.at[pl.ds(ebase, CHUNK * HW)])
        return 0

    lax.fori_loop(0, NCHUNK, chunk_body, 0)


def kernel(x, T):
    b, c, h, w = x.shape
    x1 = x.reshape(R * HW)
    t1 = T.reshape(R)
    mesh = plsc.VectorSubcoreMesh(core_axis_name="c", subcore_axis_name="s")
    out = pl.kernel(
        _body,
        out_type=jax.ShapeDtypeStruct((R * HW,), jnp.float32),
        mesh=mesh,
        scratch_types=[
            pltpu.VMEM((ROWS_PER_W + 16,), jnp.float32),
            pltpu.VMEM((CHUNK * HW + 16,), jnp.float32),
            pltpu.VMEM((32,), jnp.float32),
            pltpu.VMEM((32,), jnp.int32),
        ],
    )(x1, t1)
    return out.reshape(b, c, h, w)


# speedup vs baseline: 1.2358x; 1.0221x over previous
import jax
import jax.numpy as jnp
from jax import lax
from jax.experimental import pallas as pl
from jax.experimental.pallas import tpu as pltpu
from jax.experimental.pallas import tpu_sc as plsc

H = 56
W = 56
HW = H * W
NBLK = HW // 16
R = 64 * 256
NW = 32
ROWS_PER_W = R // NW
CHUNK = 8
NCHUNK = ROWS_PER_W // CHUNK
NHALF = NCHUNK // 2
HALF = 3


def _body(x_hbm, t_hbm, out_hbm, tbuf, buf_a, buf_b, redf, redi,
          semi_a, semi_b, semo_a, semo_b):
    cid = lax.axis_index("c")
    sid = lax.axis_index("s")
    wid = sid * 2 + cid
    base = wid * ROWS_PER_W

    lane = lax.iota(jnp.int32, 16)
    zerov = jnp.zeros((16,), jnp.float32)

    def start_in(ci, buf, sem):
        ebase = (base + ci * CHUNK) * HW
        pltpu.async_copy(x_hbm.at[pl.ds(ebase, CHUNK * HW)],
                         buf.at[pl.ds(0, CHUNK * HW)], sem)

    def wait_in(buf, sem):
        pltpu.make_async_copy(x_hbm.at[pl.ds(0, CHUNK * HW)],
                              buf.at[pl.ds(0, CHUNK * HW)], sem).wait()

    def start_out(ci, buf, sem):
        ebase = (base + ci * CHUNK) * HW
        pltpu.async_copy(buf.at[pl.ds(0, CHUNK * HW)],
                         out_hbm.at[pl.ds(ebase, CHUNK * HW)], sem)

    def wait_out(buf, sem):
        pltpu.make_async_copy(buf.at[pl.ds(0, CHUNK * HW)],
                              out_hbm.at[pl.ds(0, CHUNK * HW)], sem).wait()

    start_in(0, buf_a, semi_a)
    start_in(1, buf_b, semi_b)
    pltpu.sync_copy(t_hbm.at[pl.ds(base, ROWS_PER_W)],
                    tbuf.at[pl.ds(0, ROWS_PER_W)])

    redf[pl.ds(16, 16)] = jnp.full((16,), -jnp.inf, jnp.float32)
    redi[pl.ds(16, 16)] = jnp.full((16,), HW, jnp.int32)

    def process_row(buf, roff):
        def amax_body(j, carry):
            m, bj = carry
            v = buf[pl.ds(roff + j * 16, 16)]
            gt = v > m
            return jnp.where(gt, v, m), jnp.where(gt, j, bj)

        m0 = jnp.full((16,), -jnp.inf, jnp.float32)
        b0 = jnp.zeros((16,), jnp.int32)
        m, bj = lax.fori_loop(0, NBLK, amax_body, (m0, b0), unroll=14)

        redf[pl.ds(0, 16)] = m
        redi[pl.ds(0, 16)] = bj * 16 + lane
        for sh in (8, 4, 2, 1):
            am = redf[pl.ds(0, 16)]
            ai = redi[pl.ds(0, 16)]
            bm = redf[pl.ds(sh, 16)]
            bi = redi[pl.ds(sh, 16)]
            better = (bm > am) | ((bm == am) & (bi < ai))
            redf[pl.ds(0, 16)] = jnp.where(better, bm, am)
            redi[pl.ds(0, 16)] = jnp.where(better, bi, ai)
        idx = redi[pl.ds(0, 16)][0]

        mh = idx // W
        mw = idx - mh * W
        h1 = jnp.maximum(mh - HALF, 0)
        h2 = jnp.minimum(mh + HALF, H - 1)
        w1 = jnp.maximum(mw - HALF, 0)
        w2 = jnp.minimum(mw + HALF, W - 1)
        area = (h2 - h1 + 1) * (w2 - w1 + 1)
        area_v = jnp.full((16,), 1.0, jnp.float32) * area.astype(jnp.float32)
        lamv = jnp.float32(HW) / (jnp.float32(HW) - area_v)

        def scale_body(j, _):
            sl = pl.ds(roff + j * 16, 16)
            buf[sl] = buf[sl] * lamv
            return 0

        lax.fori_loop(0, NBLK, scale_body, 0, unroll=14)

        msk = lane <= (w2 - w1)

        def zero_body(hr, _):
            sl = pl.ds(roff + hr * W + w1, 16)
            buf[sl] = jnp.where(msk, zerov, buf[sl])
            return 0

        lax.fori_loop(h1, h2 + 1, zero_body, 0)

    def compute(buf, ci):
        tv = tbuf[pl.ds(ci * CHUNK, 16)]
        for r in range(CHUNK):
            t = tv[r]

            @pl.when(t != 0.0)
            def _(roff=r * HW):
                process_row(buf, roff)

    def pair_body(i, _):
        g0 = 2 * i
        g1 = g0 + 1

        wait_in(buf_a, semi_a)
        compute(buf_a, g0)
        start_out(g0, buf_a, semo_a)

        wait_in(buf_b, semi_b)
        compute(buf_b, g1)
        start_out(g1, buf_b, semo_b)

        @pl.when(i < NHALF - 1)
        def _():
            wait_out(buf_a, semo_a)
            start_in(g0 + 2, buf_a, semi_a)
            wait_out(buf_b, semo_b)
            start_in(g1 + 2, buf_b, semi_b)

        return 0

    lax.fori_loop(0, NHALF, pair_body, 0)
    wait_out(buf_a, semo_a)
    wait_out(buf_b, semo_b)


def kernel(x, T):
    b, c, h, w = x.shape
    x1 = x.reshape(R * HW)
    t1 = T.reshape(R)
    mesh = plsc.VectorSubcoreMesh(core_axis_name="c", subcore_axis_name="s")
    out = pl.kernel(
        _body,
        out_type=jax.ShapeDtypeStruct((R * HW,), jnp.float32),
        mesh=mesh,
        scratch_types=[
            pltpu.VMEM((ROWS_PER_W + 16,), jnp.float32),
            pltpu.VMEM((CHUNK * HW + 16,), jnp.float32),
            pltpu.VMEM((CHUNK * HW + 16,), jnp.float32),
            pltpu.VMEM((32,), jnp.float32),
            pltpu.VMEM((32,), jnp.int32),
            pltpu.SemaphoreType.DMA,
            pltpu.SemaphoreType.DMA,
            pltpu.SemaphoreType.DMA,
            pltpu.SemaphoreType.DMA,
        ],
    )(x1, t1)
    return out.reshape(b, c, h, w)
